# baseline (speedup 1.0000x reference)
import functools

import jax
import jax.numpy as jnp
from jax import lax
from jax.experimental import pallas as pl
from jax.experimental.pallas import tpu as pltpu
from jax.experimental.pallas import tpu_sc as plsc

_D = 128
_B = 16384
_NC = 2
_NS = 16
_NW = _NC * _NS
_CH = 128
_BLOCK_B = 8192


def _sc_gather(idx, table):
    nrows = idx.shape[0]
    bpw = nrows // _NW
    nch = bpw // _CH
    mesh = plsc.VectorSubcoreMesh(
        core_axis_name="c", subcore_axis_name="s",
        num_cores=_NC, num_subcores=_NS)

    @functools.partial(
        pl.kernel,
        out_type=jax.ShapeDtypeStruct((nrows, _D), jnp.float32),
        mesh=mesh,
        scratch_types=[
            pltpu.VMEM((bpw,), jnp.int32),
            pltpu.VMEM((bpw, _D), jnp.float32),
            pltpu.SemaphoreType.DMA,
            pltpu.SemaphoreType.DMA,
        ],
    )
    def gather_kernel(idx_hbm, table_hbm, out_hbm, idx_v, rows_v, sem, wsem):
        wid = lax.axis_index("s") * _NC + lax.axis_index("c")
        base = wid * bpw
        pltpu.sync_copy(idx_hbm.at[pl.ds(base, bpw)], idx_v)
        copies = [
            pltpu.async_copy(
                table_hbm.at[idx_v.at[pl.ds(j * _CH, _CH)]],
                rows_v.at[pl.ds(j * _CH, _CH), :],
                sem,
            )
            for j in range(nch)
        ]
        writes = []
        for j, c in enumerate(copies):
            c.wait()
            writes.append(pltpu.async_copy(
                rows_v.at[pl.ds(j * _CH, _CH), :],
                out_hbm.at[pl.ds(base + j * _CH, _CH), :],
                wsem,
            ))
        for w in writes:
            w.wait()

    return gather_kernel(idx, table)


def _mlp_body(emb_ref, w1_ref, b1_ref, w2_ref, b2_ref, out_ref):
    x = emb_ref[...]
    h = jnp.dot(x, w1_ref[...], preferred_element_type=jnp.float32)
    h = jnp.maximum(h + b1_ref[...], 0.0)
    o = jnp.dot(h, w2_ref[...], preferred_element_type=jnp.float32)
    o = o + b2_ref[...]
    nsq = jnp.sum(o * o, axis=1, keepdims=True)
    out_ref[...] = o * lax.rsqrt(jnp.maximum(nsq, 1e-24))


def _tc_mlp(emb, w1, b1, w2, b2):
    nrows = emb.shape[0]
    return pl.pallas_call(
        _mlp_body,
        grid=(nrows // _BLOCK_B,),
        in_specs=[
            pl.BlockSpec((_BLOCK_B, _D), lambda i: (i, 0)),
            pl.BlockSpec((_D, 2 * _D), lambda i: (0, 0)),
            pl.BlockSpec((1, 2 * _D), lambda i: (0, 0)),
            pl.BlockSpec((2 * _D, _D), lambda i: (0, 0)),
            pl.BlockSpec((1, _D), lambda i: (0, 0)),
        ],
        out_specs=pl.BlockSpec((_BLOCK_B, _D), lambda i: (i, 0)),
        out_shape=jax.ShapeDtypeStruct((nrows, _D), jnp.float32),
    )(emb, w1, b1, w2, b2)


def kernel(item_ids, table, W1, b1, W2, b2):
    emb = _sc_gather(item_ids.astype(jnp.int32), table)
    return _tc_mlp(emb, W1, b1.reshape(1, -1), W2, b2.reshape(1, -1))
